# baseline (speedup 1.0000x reference)
import dataclasses
import functools

import jax
import jax.numpy as jnp
from jax import lax
from jax.experimental import pallas as pl
from jax.experimental.pallas import tpu as pltpu
from jax.experimental.pallas import tpu_sc as plsc

N = 10000
E = 320000
NPAD = 10240
NW = 32
CHUNK = 128
F1 = 144
F2 = 80
D_OUT = 64

NB = 9
EPT = 11520
SPT = 10240
RPT = NPAD // 16

_SC_PARAMS = pltpu.CompilerParams()
for _f, _v in (("needs_layout_passes", False), ("use_tc_tiling_on_sc", False)):
    if _f in pltpu.CompilerParams.__dataclass_fields__:
        _SC_PARAMS = dataclasses.replace(_SC_PARAMS, **{_f: _v})


def _proj1_body(x_ref, w_ref, att_ref, haug_ref, asad_ref):
    h = lax.dot_general(x_ref[...], w_ref[...], (((1,), (1,)), ((), ())),
                        preferred_element_type=jnp.float32)
    asad_ref[...] = lax.dot_general(att_ref[...], h, (((1,), (1,)), ((), ())),
                                    preferred_element_type=jnp.float32)
    as_col = lax.dot_general(h, att_ref[...], (((1,), (1,)), ((), ())),
                             preferred_element_type=jnp.float32)[:, 0:1]
    io = lax.broadcasted_iota(jnp.int32, (h.shape[0], 16), 1)
    tail = (io == 0).astype(jnp.float32) + jnp.where(io == 1, as_col, 0.0)
    haug_ref[...] = jnp.concatenate([h, tail], axis=1)


def _proj1(xpad, W1, att1):
    grid = NPAD // 1024
    return pl.pallas_call(
        _proj1_body,
        grid=(grid,),
        in_specs=[
            pl.BlockSpec((1024, 128), lambda i: (i, 0)),
            pl.BlockSpec((128, 128), lambda i: (0, 0)),
            pl.BlockSpec((2, 128), lambda i: (0, 0)),
        ],
        out_specs=[
            pl.BlockSpec((1024, F1), lambda i: (i, 0)),
            pl.BlockSpec((2, 1024), lambda i: (0, i)),
        ],
        out_shape=[
            jax.ShapeDtypeStruct((NPAD, F1), jnp.float32),
            jax.ShapeDtypeStruct((2, NPAD), jnp.float32),
        ],
    )(xpad, W1, att1)


def _gru_body(acc_ref, b1_ref, wih_ref, bih_ref, bhh_ref, w2_ref, att2_ref,
              hn_ref, haug2_ref, asad2_ref):
    A = acc_ref[0] + acc_ref[1]
    den = A[:, 128:129]
    h = jnp.maximum(A[:, :128] / (den + 1e-16) + b1_ref[...], 0.0)
    gi = lax.dot_general(h, wih_ref[...], (((1,), (1,)), ((), ())),
                         preferred_element_type=jnp.float32) + bih_ref[...]
    r = jax.nn.sigmoid(gi[:, :128] + bhh_ref[:, :128])
    z = jax.nn.sigmoid(gi[:, 128:256] + bhh_ref[:, 128:256])
    n = jnp.tanh(gi[:, 256:] + r * bhh_ref[:, 256:])
    hn = (1.0 - z) * n
    hn_ref[...] = hn
    h2 = lax.dot_general(hn, w2_ref[...], (((1,), (1,)), ((), ())),
                         preferred_element_type=jnp.float32)
    asad2_ref[...] = lax.dot_general(att2_ref[...], h2, (((1,), (1,)), ((), ())),
                                     preferred_element_type=jnp.float32)
    as_col = lax.dot_general(h2, att2_ref[...], (((1,), (1,)), ((), ())),
                             preferred_element_type=jnp.float32)[:, 0:1]
    io = lax.broadcasted_iota(jnp.int32, (h2.shape[0], 16), 1)
    tail = (io == 0).astype(jnp.float32) + jnp.where(io == 1, as_col, 0.0)
    haug2_ref[...] = jnp.concatenate([h2, tail], axis=1)


def _gru(acc1, b1, w_ih, b_ih, b_hh, W2, att2):
    grid = NPAD // 1024
    return pl.pallas_call(
        _gru_body,
        grid=(grid,),
        in_specs=[
            pl.BlockSpec((2, 1024, F1), lambda i: (0, i, 0)),
            pl.BlockSpec((1, 128), lambda i: (0, 0)),
            pl.BlockSpec((384, 128), lambda i: (0, 0)),
            pl.BlockSpec((1, 384), lambda i: (0, 0)),
            pl.BlockSpec((1, 384), lambda i: (0, 0)),
            pl.BlockSpec((64, 128), lambda i: (0, 0)),
            pl.BlockSpec((2, 64), lambda i: (0, 0)),
        ],
        out_specs=[
            pl.BlockSpec((1024, 128), lambda i: (i, 0)),
            pl.BlockSpec((1024, F2), lambda i: (i, 0)),
            pl.BlockSpec((2, 1024), lambda i: (0, i)),
        ],
        out_shape=[
            jax.ShapeDtypeStruct((NPAD, 128), jnp.float32),
            jax.ShapeDtypeStruct((NPAD, F2), jnp.float32),
            jax.ShapeDtypeStruct((2, NPAD), jnp.float32),
        ],
    )(acc1, b1, w_ih, b_ih, b_hh, W2, att2)


def _fin_body(acc_ref, b2_ref, z_ref):
    A = acc_ref[0] + acc_ref[1]
    den = A[:, 64:65]
    z_ref[...] = jnp.maximum(A[:, :64] / (den + 1e-16) + b2_ref[...], 0.0)


def _fin(acc2, b2):
    grid = NPAD // 1024
    return pl.pallas_call(
        _fin_body,
        grid=(grid,),
        in_specs=[
            pl.BlockSpec((2, 1024, F2), lambda i: (0, i, 0)),
            pl.BlockSpec((1, 64), lambda i: (0, 0)),
        ],
        out_specs=pl.BlockSpec((1024, 64), lambda i: (i, 0)),
        out_shape=jax.ShapeDtypeStruct((NPAD, 64), jnp.float32),
    )(acc2, b2)


def _sc_gat(haug, asad, src4, dst4, F, chunk, bi, nblk):
    mesh = plsc.VectorSubcoreMesh(core_axis_name="c", subcore_axis_name="s")
    as_col = F - 15
    nbuf, ah = 3, 2
    zb = 128 if RPT % 128 == 0 and chunk >= 128 else 40

    @functools.partial(
        pl.kernel,
        out_type=jax.ShapeDtypeStruct((2, NPAD, F), jnp.float32),
        mesh=mesh,
        compiler_params=_SC_PARAMS,
        scratch_types=[
            pltpu.VMEM((2, bi, chunk), jnp.int32),
            pltpu.VMEM((2, bi, chunk), jnp.int32),
            pltpu.VMEM((NPAD,), jnp.float32),
            pltpu.VMEM((nbuf, chunk, F), jnp.float32),
            pltpu.VMEM((nbuf, chunk), jnp.float32),
            pltpu.VMEM_SHARED((NPAD, F), jnp.float32),
        ] + [pltpu.SemaphoreType.DMA] * (2 * nbuf + 2),
    )
    def k(haug_hbm, asad_hbm, src_hbm, dst_hbm, out_hbm,
          si_v, di_v, ad_v, rows_v, w_v, acc_sh, *sems):
        gsem = sems[:nbuf]
        ssem = sems[nbuf:2 * nbuf]
        isem, fsem = sems[2 * nbuf:]
        c = lax.axis_index("c")
        s = lax.axis_index("s")
        wid = s * 2 + c
        pltpu.sync_copy(asad_hbm.at[1], ad_v)

        @pl.loop(0, zb)
        def _(r):
            @pl.loop(0, F // 16)
            def _(g):
                rows_v[0, r, pl.ds(g * 16, 16)] = jnp.zeros((16,), jnp.float32)

        zcp = [pltpu.async_copy(rows_v.at[0, pl.ds(0, zb)],
                                acc_sh.at[pl.ds(s * RPT + i * zb, zb)], fsem)
               for i in range(RPT // zb)]
        for cp in zcp:
            cp.wait()

        plsc.subcore_barrier()

        pltpu.sync_copy(src_hbm.at[wid, 0], si_v.at[0])
        pltpu.sync_copy(dst_hbm.at[wid, 0], di_v.at[0])

        def chunk_body(p, kk):
            b = kk % nbuf
            bb = jnp.full((16,), b, jnp.int32)
            for g in range(chunk // 16):
                ridx = g * 16 + lax.broadcasted_iota(jnp.int32, (16,), 0)
                a_s = plsc.load_gather(
                    rows_v, [bb, ridx, jnp.full((16,), as_col, jnp.int32)])
                a_d = plsc.load_gather(ad_v, [di_v[p, kk, pl.ds(g * 16, 16)]])
                e = a_s + a_d
                e = jnp.where(e > 0, e, 0.2 * e)
                w_v[b, pl.ds(g * 16, 16)] = jnp.exp(e)

            @plsc.parallel_loop(0, chunk, unroll=4)
            def _(r):
                wr = plsc.load_gather(w_v, [bb, jnp.full((16,), r, jnp.int32)])
                for g in range(F // 16):
                    sl = pl.ds(g * 16, 16)
                    rows_v[b, r, sl] = rows_v[b, r, sl] * wr

            return pltpu.async_copy(rows_v.at[b], acc_sh.at[di_v.at[p, kk]],
                                    ssem[b], add=True)

        @pl.loop(0, nblk)
        def _(bl):
            p = bl % 2
            icp0 = pltpu.async_copy(src_hbm.at[wid, bl + 1], si_v.at[1 - p], isem)
            icp1 = pltpu.async_copy(dst_hbm.at[wid, bl + 1], di_v.at[1 - p], isem)
            gcp = [None] * bi
            scp = [None] * bi
            for a in range(ah):
                gcp[a] = pltpu.async_copy(haug_hbm.at[si_v.at[p, a]],
                                          rows_v.at[a % nbuf], gsem[a % nbuf])
            for kk in range(bi):
                gcp[kk].wait()
                nxt = kk + ah
                if nxt < bi:
                    if nxt - nbuf >= 0:
                        scp[nxt - nbuf].wait()
                    gcp[nxt] = pltpu.async_copy(haug_hbm.at[si_v.at[p, nxt]],
                                                rows_v.at[nxt % nbuf],
                                                gsem[nxt % nbuf])
                scp[kk] = chunk_body(p, kk)
            for kk in range(max(0, bi - nbuf), bi):
                scp[kk].wait()
            icp0.wait()
            icp1.wait()

        plsc.subcore_barrier()

        fcp = [pltpu.async_copy(acc_sh.at[pl.ds(s * RPT + i * zb, zb)],
                                out_hbm.at[c].at[pl.ds(s * RPT + i * zb, zb)],
                                fsem)
               for i in range(RPT // zb)]
        for cp in fcp:
            cp.wait()

    return k(haug, asad, src4, dst4)


def _sc_score(z, e0r, e1r):
    mesh = plsc.VectorSubcoreMesh(core_axis_name="c", subcore_axis_name="s")
    ks = SPT // CHUNK
    sbi = 16
    nbuf, ah = 4, 2

    @functools.partial(
        pl.kernel,
        out_type=jax.ShapeDtypeStruct((NW * SPT,), jnp.float32),
        mesh=mesh,
        compiler_params=_SC_PARAMS,
        scratch_types=[
            pltpu.VMEM((ks, CHUNK), jnp.int32),
            pltpu.VMEM((ks, CHUNK), jnp.int32),
            pltpu.VMEM((nbuf, CHUNK, D_OUT), jnp.float32),
            pltpu.VMEM((nbuf, CHUNK, D_OUT), jnp.float32),
            pltpu.VMEM((nbuf, CHUNK), jnp.float32),
            pltpu.VMEM_SHARED((NPAD, D_OUT), jnp.float32),
        ] + [pltpu.SemaphoreType.DMA] * (3 * nbuf),
    )
    def k(z_hbm, e0_hbm, e1_hbm, out_hbm, i0_v, i1_v, za_v, zb_v, s_v, z_sh,
          *sems):
        gsa = sems[:nbuf]
        gsb = sems[nbuf:2 * nbuf]
        osem = sems[2 * nbuf:]
        c = lax.axis_index("c")
        s = lax.axis_index("s")
        wid = s * 2 + c
        pltpu.sync_copy(e0_hbm.at[wid], i0_v)
        pltpu.sync_copy(e1_hbm.at[wid], i1_v)
        lane0 = lax.broadcasted_iota(jnp.int32, (16,), 0) == 0

        tpr = NPAD // 16
        pltpu.sync_copy(z_hbm.at[pl.ds(s * tpr, tpr)],
                        z_sh.at[pl.ds(s * tpr, tpr)])
        plsc.subcore_barrier()

        def compute(t, b):
            @plsc.parallel_loop(0, CHUNK, unroll=4)
            def _(r):
                acc = za_v[b, r, pl.ds(0, 16)] * zb_v[b, r, pl.ds(0, 16)]
                for g in range(1, D_OUT // 16):
                    sl = pl.ds(g * 16, 16)
                    acc = acc + za_v[b, r, sl] * zb_v[b, r, sl]
                tot = jnp.sum(acc)
                plsc.store_scatter(s_v, [jnp.full((16,), b, jnp.int32),
                                         jnp.full((16,), r, jnp.int32)],
                                   jnp.full((16,), tot, jnp.float32), mask=lane0)
            return pltpu.async_copy(
                s_v.at[b], out_hbm.at[pl.ds((wid * ks + t) * CHUNK, CHUNK)],
                osem[b])

        def gather_pair(t, b):
            return (pltpu.async_copy(z_sh.at[i0_v.at[t]], za_v.at[b], gsa[b]),
                    pltpu.async_copy(z_sh.at[i1_v.at[t]], zb_v.at[b], gsb[b]))

        @pl.loop(0, ks // sbi)
        def _(bl):
            t0 = bl * sbi
            gcp = [None] * sbi
            ocp = [None] * sbi
            for a in range(ah):
                gcp[a] = gather_pair(t0 + a, a % nbuf)
            for kk in range(sbi):
                b = kk % nbuf
                gcp[kk][0].wait()
                gcp[kk][1].wait()
                nxt = kk + ah
                if nxt < sbi:
                    if nxt - nbuf >= 0:
                        ocp[nxt - nbuf].wait()
                    gcp[nxt] = gather_pair(t0 + nxt, nxt % nbuf)
                ocp[kk] = compute(t0 + kk, b)
            for kk in range(max(0, sbi - nbuf), sbi):
                ocp[kk].wait()

    return k(z, e0r, e1r)


def _pad_edges(a, per_tile):
    a = a.astype(jnp.int32)
    a = jnp.concatenate([a, jnp.full((NW * per_tile - a.shape[0],), N, jnp.int32)])
    return a.reshape(per_tile, NW).T


def kernel(x, edge_index, edge_label_index, W1, att_src1, att_dst1, b1,
           w_ih, w_hh, b_ih, b_hh, W2, att_src2, att_dst2, b2):
    sl = jnp.arange(N, dtype=jnp.int32)
    src = _pad_edges(jnp.concatenate([edge_index[0].astype(jnp.int32), sl]), EPT)
    dst = _pad_edges(jnp.concatenate([edge_index[1].astype(jnp.int32), sl]), EPT)
    src41 = src.reshape(NW, 16, 15, 48)
    dst41 = dst.reshape(NW, 16, 15, 48)
    src42 = src.reshape(NW, 10, 9, 128)
    dst42 = dst.reshape(NW, 10, 9, 128)
    e0r = _pad_edges(edge_label_index[0], SPT).reshape(NW, SPT // CHUNK, CHUNK)
    e1r = _pad_edges(edge_label_index[1], SPT).reshape(NW, SPT // CHUNK, CHUNK)

    xpad = jnp.zeros((NPAD, 128), jnp.float32).at[:N].set(x)
    att1 = jnp.stack([att_src1, att_dst1])
    att2 = jnp.stack([att_src2, att_dst2])

    haug, asad = _proj1(xpad, W1, att1)
    acc1 = _sc_gat(haug, asad, src41, dst41, F1, 48, 15, 15)
    hn, haug2, asad2 = _gru(acc1, b1.reshape(1, 128), w_ih, b_ih.reshape(1, 384),
                            b_hh.reshape(1, 384), W2, att2)
    acc2 = _sc_gat(haug2, asad2, src42, dst42, F2, 128, 9, 9)
    zfin = _fin(acc2, b2.reshape(1, 64))
    scores = _sc_score(zfin, e0r, e1r)
    scores = scores.reshape(NW, SPT).T.reshape(-1)[:E]
    return scores, hn[:N][None]


# speedup vs baseline: 1.7825x; 1.7825x over previous
import dataclasses
import functools

import jax
import jax.numpy as jnp
from jax import lax
from jax.experimental import pallas as pl
from jax.experimental.pallas import tpu as pltpu
from jax.experimental.pallas import tpu_sc as plsc

N = 10000
E = 320000
NPAD = 10240
NW = 32
CHUNK = 128
F1 = 144
F2 = 80
D_OUT = 64

NB = 9
EPT = 11520
SPT = 10240
RPT = NPAD // 16

_SC_PARAMS = pltpu.CompilerParams()
for _f, _v in (("needs_layout_passes", False), ("use_tc_tiling_on_sc", False)):
    if _f in pltpu.CompilerParams.__dataclass_fields__:
        _SC_PARAMS = dataclasses.replace(_SC_PARAMS, **{_f: _v})


def _proj1_body(x_ref, w_ref, att_ref, haug_ref, asad_ref):
    h = lax.dot_general(x_ref[...], w_ref[...], (((1,), (1,)), ((), ())),
                        preferred_element_type=jnp.float32)
    asad_ref[...] = lax.dot_general(att_ref[...], h, (((1,), (1,)), ((), ())),
                                    preferred_element_type=jnp.float32)
    as_col = lax.dot_general(h, att_ref[...], (((1,), (1,)), ((), ())),
                             preferred_element_type=jnp.float32)[:, 0:1]
    io = lax.broadcasted_iota(jnp.int32, (h.shape[0], 16), 1)
    tail = (io == 0).astype(jnp.float32) + jnp.where(io == 1, as_col, 0.0)
    haug_ref[...] = jnp.concatenate([h, tail], axis=1)


def _proj1(xpad, W1, att1):
    grid = NPAD // 1024
    return pl.pallas_call(
        _proj1_body,
        grid=(grid,),
        in_specs=[
            pl.BlockSpec((1024, 128), lambda i: (i, 0)),
            pl.BlockSpec((128, 128), lambda i: (0, 0)),
            pl.BlockSpec((2, 128), lambda i: (0, 0)),
        ],
        out_specs=[
            pl.BlockSpec((1024, F1), lambda i: (i, 0)),
            pl.BlockSpec((2, 1024), lambda i: (0, i)),
        ],
        out_shape=[
            jax.ShapeDtypeStruct((NPAD, F1), jnp.float32),
            jax.ShapeDtypeStruct((2, NPAD), jnp.float32),
        ],
    )(xpad, W1, att1)


def _gru_body(acc_ref, b1_ref, wih_ref, bih_ref, bhh_ref, w2_ref, att2_ref,
              hn_ref, haug2_ref, asad2_ref):
    A = acc_ref[0] + acc_ref[1]
    den = A[:, 128:129]
    h = jnp.maximum(A[:, :128] / (den + 1e-16) + b1_ref[...], 0.0)
    gi = lax.dot_general(h, wih_ref[...], (((1,), (1,)), ((), ())),
                         preferred_element_type=jnp.float32) + bih_ref[...]
    r = jax.nn.sigmoid(gi[:, :128] + bhh_ref[:, :128])
    z = jax.nn.sigmoid(gi[:, 128:256] + bhh_ref[:, 128:256])
    n = jnp.tanh(gi[:, 256:] + r * bhh_ref[:, 256:])
    hn = (1.0 - z) * n
    hn_ref[...] = hn
    h2 = lax.dot_general(hn, w2_ref[...], (((1,), (1,)), ((), ())),
                         preferred_element_type=jnp.float32)
    asad2_ref[...] = lax.dot_general(att2_ref[...], h2, (((1,), (1,)), ((), ())),
                                     preferred_element_type=jnp.float32)
    as_col = lax.dot_general(h2, att2_ref[...], (((1,), (1,)), ((), ())),
                             preferred_element_type=jnp.float32)[:, 0:1]
    io = lax.broadcasted_iota(jnp.int32, (h2.shape[0], 16), 1)
    tail = (io == 0).astype(jnp.float32) + jnp.where(io == 1, as_col, 0.0)
    haug2_ref[...] = jnp.concatenate([h2, tail], axis=1)


def _gru(acc1, b1, w_ih, b_ih, b_hh, W2, att2):
    grid = NPAD // 1024
    return pl.pallas_call(
        _gru_body,
        grid=(grid,),
        in_specs=[
            pl.BlockSpec((2, 1024, F1), lambda i: (0, i, 0)),
            pl.BlockSpec((1, 128), lambda i: (0, 0)),
            pl.BlockSpec((384, 128), lambda i: (0, 0)),
            pl.BlockSpec((1, 384), lambda i: (0, 0)),
            pl.BlockSpec((1, 384), lambda i: (0, 0)),
            pl.BlockSpec((64, 128), lambda i: (0, 0)),
            pl.BlockSpec((2, 64), lambda i: (0, 0)),
        ],
        out_specs=[
            pl.BlockSpec((1024, 128), lambda i: (i, 0)),
            pl.BlockSpec((1024, F2), lambda i: (i, 0)),
            pl.BlockSpec((2, 1024), lambda i: (0, i)),
        ],
        out_shape=[
            jax.ShapeDtypeStruct((NPAD, 128), jnp.float32),
            jax.ShapeDtypeStruct((NPAD, F2), jnp.float32),
            jax.ShapeDtypeStruct((2, NPAD), jnp.float32),
        ],
    )(acc1, b1, w_ih, b_ih, b_hh, W2, att2)


def _fin_body(acc_ref, b2_ref, z_ref):
    A = acc_ref[0] + acc_ref[1]
    den = A[:, 64:65]
    z_ref[...] = jnp.maximum(A[:, :64] / (den + 1e-16) + b2_ref[...], 0.0)


def _fin(acc2, b2):
    grid = NPAD // 1024
    return pl.pallas_call(
        _fin_body,
        grid=(grid,),
        in_specs=[
            pl.BlockSpec((2, 1024, F2), lambda i: (0, i, 0)),
            pl.BlockSpec((1, 64), lambda i: (0, 0)),
        ],
        out_specs=pl.BlockSpec((1024, 64), lambda i: (i, 0)),
        out_shape=jax.ShapeDtypeStruct((NPAD, 64), jnp.float32),
    )(acc2, b2)


def _sc_gat(haug, asad, src4, dst4, F, chunk, bi, nblk, nbuf, ah):
    mesh = plsc.VectorSubcoreMesh(core_axis_name="c", subcore_axis_name="s")
    as_col = F - 15
    zb = 128 if chunk >= 128 else 64 if chunk >= 64 else 40

    @functools.partial(
        pl.kernel,
        out_type=jax.ShapeDtypeStruct((2, NPAD, F), jnp.float32),
        mesh=mesh,
        compiler_params=_SC_PARAMS,
        scratch_types=[
            pltpu.VMEM((2, bi, chunk), jnp.int32),
            pltpu.VMEM((2, bi, chunk), jnp.int32),
            pltpu.VMEM((NPAD,), jnp.float32),
            pltpu.VMEM((nbuf, chunk, F), jnp.float32),
            pltpu.VMEM((nbuf, chunk), jnp.float32),
            pltpu.VMEM_SHARED((NPAD, F), jnp.float32),
        ] + [pltpu.SemaphoreType.DMA] * (2 * nbuf + 2),
    )
    def k(haug_hbm, asad_hbm, src_hbm, dst_hbm, out_hbm,
          si_v, di_v, ad_v, rows_v, w_v, acc_sh, *sems):
        gsem = sems[:nbuf]
        ssem = sems[nbuf:2 * nbuf]
        isem, fsem = sems[2 * nbuf:]
        c = lax.axis_index("c")
        s = lax.axis_index("s")
        wid = s * 2 + c
        pltpu.sync_copy(asad_hbm.at[1], ad_v)

        @pl.loop(0, zb)
        def _(r):
            @pl.loop(0, F // 16)
            def _(g):
                rows_v[0, r, pl.ds(g * 16, 16)] = jnp.zeros((16,), jnp.float32)

        zcp = [pltpu.async_copy(rows_v.at[0, pl.ds(0, zb)],
                                acc_sh.at[pl.ds(s * RPT + i * zb, zb)], fsem)
               for i in range(RPT // zb)]
        for cp in zcp:
            cp.wait()

        plsc.subcore_barrier()

        pltpu.sync_copy(src_hbm.at[wid, 0], si_v.at[0])
        pltpu.sync_copy(dst_hbm.at[wid, 0], di_v.at[0])

        def chunk_body(p, kk):
            b = kk % nbuf
            bb = jnp.full((16,), b, jnp.int32)
            for g in range(chunk // 16):
                ridx = g * 16 + lax.broadcasted_iota(jnp.int32, (16,), 0)
                a_s = plsc.load_gather(
                    rows_v, [bb, ridx, jnp.full((16,), as_col, jnp.int32)])
                a_d = plsc.load_gather(ad_v, [di_v[p, kk, pl.ds(g * 16, 16)]])
                e = a_s + a_d
                e = jnp.where(e > 0, e, 0.2 * e)
                w_v[b, pl.ds(g * 16, 16)] = jnp.exp(e)

            @plsc.parallel_loop(0, chunk, unroll=4)
            def _(r):
                wr = plsc.load_gather(w_v, [bb, jnp.full((16,), r, jnp.int32)])
                for g in range(F // 16):
                    sl = pl.ds(g * 16, 16)
                    rows_v[b, r, sl] = rows_v[b, r, sl] * wr

            return pltpu.async_copy(rows_v.at[b], acc_sh.at[di_v.at[p, kk]],
                                    ssem[b], add=True)

        @pl.loop(0, nblk)
        def _(bl):
            p = bl % 2
            icp0 = pltpu.async_copy(src_hbm.at[wid, bl + 1], si_v.at[1 - p], isem)
            icp1 = pltpu.async_copy(dst_hbm.at[wid, bl + 1], di_v.at[1 - p], isem)
            gcp = [None] * bi
            scp = [None] * bi
            for a in range(ah):
                gcp[a] = pltpu.async_copy(haug_hbm.at[si_v.at[p, a]],
                                          rows_v.at[a % nbuf], gsem[a % nbuf])
            for kk in range(bi):
                gcp[kk].wait()
                nxt = kk + ah
                if nxt < bi:
                    if nxt - nbuf >= 0:
                        scp[nxt - nbuf].wait()
                    gcp[nxt] = pltpu.async_copy(haug_hbm.at[si_v.at[p, nxt]],
                                                rows_v.at[nxt % nbuf],
                                                gsem[nxt % nbuf])
                scp[kk] = chunk_body(p, kk)
            for kk in range(max(0, bi - nbuf), bi):
                scp[kk].wait()
            icp0.wait()
            icp1.wait()

        plsc.subcore_barrier()

        fcp = [pltpu.async_copy(acc_sh.at[pl.ds(s * RPT + i * zb, zb)],
                                out_hbm.at[c].at[pl.ds(s * RPT + i * zb, zb)],
                                fsem)
               for i in range(RPT // zb)]
        for cp in fcp:
            cp.wait()

    return k(haug, asad, src4, dst4)


def _sc_score(z, e0r, e1r):
    mesh = plsc.VectorSubcoreMesh(core_axis_name="c", subcore_axis_name="s")
    ks = SPT // CHUNK
    sbi = 16
    nbuf, ah = 4, 2

    @functools.partial(
        pl.kernel,
        out_type=jax.ShapeDtypeStruct((NW * SPT,), jnp.float32),
        mesh=mesh,
        compiler_params=_SC_PARAMS,
        scratch_types=[
            pltpu.VMEM((ks, CHUNK), jnp.int32),
            pltpu.VMEM((ks, CHUNK), jnp.int32),
            pltpu.VMEM((nbuf, CHUNK, D_OUT), jnp.float32),
            pltpu.VMEM((nbuf, CHUNK, D_OUT), jnp.float32),
            pltpu.VMEM((nbuf, CHUNK), jnp.float32),
            pltpu.VMEM_SHARED((NPAD, D_OUT), jnp.float32),
        ] + [pltpu.SemaphoreType.DMA] * (3 * nbuf),
    )
    def k(z_hbm, e0_hbm, e1_hbm, out_hbm, i0_v, i1_v, za_v, zb_v, s_v, z_sh,
          *sems):
        gsa = sems[:nbuf]
        gsb = sems[nbuf:2 * nbuf]
        osem = sems[2 * nbuf:]
        c = lax.axis_index("c")
        s = lax.axis_index("s")
        wid = s * 2 + c
        pltpu.sync_copy(e0_hbm.at[wid], i0_v)
        pltpu.sync_copy(e1_hbm.at[wid], i1_v)
        lane0 = lax.broadcasted_iota(jnp.int32, (16,), 0) == 0

        tpr = NPAD // 16
        pltpu.sync_copy(z_hbm.at[pl.ds(s * tpr, tpr)],
                        z_sh.at[pl.ds(s * tpr, tpr)])
        plsc.subcore_barrier()

        def compute(t, b):
            @plsc.parallel_loop(0, CHUNK, unroll=4)
            def _(r):
                acc = za_v[b, r, pl.ds(0, 16)] * zb_v[b, r, pl.ds(0, 16)]
                for g in range(1, D_OUT // 16):
                    sl = pl.ds(g * 16, 16)
                    acc = acc + za_v[b, r, sl] * zb_v[b, r, sl]
                tot = jnp.sum(acc)
                plsc.store_scatter(s_v, [jnp.full((16,), b, jnp.int32),
                                         jnp.full((16,), r, jnp.int32)],
                                   jnp.full((16,), tot, jnp.float32), mask=lane0)
            return pltpu.async_copy(
                s_v.at[b], out_hbm.at[pl.ds((wid * ks + t) * CHUNK, CHUNK)],
                osem[b])

        def gather_pair(t, b):
            return (pltpu.async_copy(z_sh.at[i0_v.at[t]], za_v.at[b], gsa[b]),
                    pltpu.async_copy(z_sh.at[i1_v.at[t]], zb_v.at[b], gsb[b]))

        @pl.loop(0, ks // sbi)
        def _(bl):
            t0 = bl * sbi
            gcp = [None] * sbi
            ocp = [None] * sbi
            for a in range(ah):
                gcp[a] = gather_pair(t0 + a, a % nbuf)
            for kk in range(sbi):
                b = kk % nbuf
                gcp[kk][0].wait()
                gcp[kk][1].wait()
                nxt = kk + ah
                if nxt < sbi:
                    if nxt - nbuf >= 0:
                        ocp[nxt - nbuf].wait()
                    gcp[nxt] = gather_pair(t0 + nxt, nxt % nbuf)
                ocp[kk] = compute(t0 + kk, b)
            for kk in range(max(0, sbi - nbuf), sbi):
                ocp[kk].wait()

    return k(z, e0r, e1r)


def _pad_edges(a, per_tile):
    a = a.astype(jnp.int32)
    a = jnp.concatenate([a, jnp.full((NW * per_tile - a.shape[0],), N, jnp.int32)])
    return a.reshape(per_tile, NW).T


def kernel(x, edge_index, edge_label_index, W1, att_src1, att_dst1, b1,
           w_ih, w_hh, b_ih, b_hh, W2, att_src2, att_dst2, b2):
    sl = jnp.arange(N, dtype=jnp.int32)
    src = _pad_edges(jnp.concatenate([edge_index[0].astype(jnp.int32), sl]), EPT)
    dst = _pad_edges(jnp.concatenate([edge_index[1].astype(jnp.int32), sl]), EPT)
    src41 = src.reshape(NW, 10, 18, 64)
    dst41 = dst.reshape(NW, 10, 18, 64)
    src42 = src.reshape(NW, 10, 9, 128)
    dst42 = dst.reshape(NW, 10, 9, 128)
    e0r = _pad_edges(edge_label_index[0], SPT).reshape(NW, SPT // CHUNK, CHUNK)
    e1r = _pad_edges(edge_label_index[1], SPT).reshape(NW, SPT // CHUNK, CHUNK)

    xpad = jnp.zeros((NPAD, 128), jnp.float32).at[:N].set(x)
    att1 = jnp.stack([att_src1, att_dst1])
    att2 = jnp.stack([att_src2, att_dst2])

    haug, asad = _proj1(xpad, W1, att1)
    acc1 = _sc_gat(haug, asad, src41, dst41, F1, 64, 18, 9, 2, 1)
    hn, haug2, asad2 = _gru(acc1, b1.reshape(1, 128), w_ih, b_ih.reshape(1, 384),
                            b_hh.reshape(1, 384), W2, att2)
    acc2 = _sc_gat(haug2, asad2, src42, dst42, F2, 128, 9, 9, 3, 2)
    zfin = _fin(acc2, b2.reshape(1, 64))
    scores = _sc_score(zfin, e0r, e1r)
    scores = scores.reshape(NW, SPT).T.reshape(-1)[:E]
    return scores, hn[:N][None]
